# Initial kernel scaffold; baseline (speedup 1.0000x reference)
import jax
import jax.numpy as jnp
from jax.experimental import pallas as pl


def kernel(x, edge_index, ptr, W1, b1, W2, b2, Wl, bl):
    raise NotImplementedError("write your pallas kernel here")


# speedup vs baseline: 9.0378x; 9.0378x over previous
import functools

import jax
import jax.numpy as jnp
from jax import lax
from jax.experimental import pallas as pl
from jax.experimental.pallas import tpu as pltpu
from jax.experimental.pallas import tpu_sc as plsc

N = 50000
E = 800000
NGRAPH = 16

NC = 2
NS = 16
NW = NC * NS
BATCH = 128
KB = 8
NB = 200
EPWP = NB * BATCH
PADE = EPWP * NW - E
NACC = 50176
RPT = NACC // NS
ZROWS = 56
CPR = RPT // ZROWS
CPO = 3128
NOUT = CPO * NS
DC = 32

R = 512
GRID = (N + R - 1) // R


def _sc_mesh():
    return plsc.VectorSubcoreMesh(core_axis_name="c", subcore_axis_name="s")


def _fill_zero(zbuf, ncols):
    zv = jnp.zeros((16,), jnp.float32)

    def zrow(i, _):
        for c0 in range(0, ncols, 16):
            zbuf[i, pl.ds(c0, 16)] = zv
        return 0

    lax.fori_loop(0, ZROWS, zrow, 0, unroll=2)


def _make_deg_kernel():
    @functools.partial(
        pl.kernel,
        out_type=jax.ShapeDtypeStruct((NC, NOUT, 16), jnp.float32),
        mesh=_sc_mesh(),
        compiler_params=pltpu.CompilerParams(use_tc_tiling_on_sc=False),
        scratch_types=[
            pltpu.VMEM((KB, BATCH), jnp.int32),
            pltpu.VMEM((BATCH, 16), jnp.float32),
            pltpu.VMEM((ZROWS, 16), jnp.float32),
            pltpu.VMEM_SHARED((NACC, 16), jnp.float32),
        ],
    )
    def deg_kernel(dst_hbm, out_hbm, dst_v, ones_v, zbuf, acc):
        c = lax.axis_index("c")
        s = lax.axis_index("s")
        u = c * NS + s

        ov = jnp.ones((16,), jnp.float32)

        def orow(i, _):
            ones_v[i, pl.ds(0, 16)] = ov
            return 0

        lax.fori_loop(0, BATCH, orow, 0, unroll=2)
        _fill_zero(zbuf, 16)

        def zcp(k, _):
            pltpu.sync_copy(zbuf, acc.at[pl.ds(s * RPT + k * ZROWS, ZROWS)])
            return 0

        lax.fori_loop(0, CPR, zcp, 0)
        plsc.subcore_barrier()

        def kblk(k, _):
            pltpu.sync_copy(dst_hbm.at[u, pl.ds(k * KB, KB)], dst_v)

            def estep(j, _):
                pltpu.sync_copy(ones_v, acc.at[dst_v.at[j]], add=True)
                return 0

            lax.fori_loop(0, KB, estep, 0)
            return 0

        lax.fori_loop(0, NB // KB, kblk, 0)
        plsc.subcore_barrier()
        pltpu.sync_copy(
            acc.at[pl.ds(s * CPO, CPO)],
            out_hbm.at[c, pl.ds(s * CPO, CPO)],
        )

    return deg_kernel


def _make_agg_kernel(nchunks):
    @functools.partial(
        pl.kernel,
        out_type=tuple(
            jax.ShapeDtypeStruct((NC, NOUT, DC), jnp.float32) for _ in range(nchunks)
        ),
        mesh=_sc_mesh(),
        compiler_params=pltpu.CompilerParams(use_tc_tiling_on_sc=False),
        scratch_types=[
            pltpu.VMEM((KB, BATCH), jnp.int32),
            pltpu.VMEM((KB, BATCH), jnp.int32),
            pltpu.VMEM((BATCH, DC), jnp.float32),
            pltpu.VMEM((BATCH, DC), jnp.float32),
            pltpu.VMEM((ZROWS, DC), jnp.float32),
            pltpu.VMEM_SHARED((NACC, DC), jnp.float32),
            pltpu.SemaphoreType.DMA,
            pltpu.SemaphoreType.DMA,
        ],
    )
    def agg_kernel(*refs):
        tables = refs[:nchunks]
        src_hbm = refs[nchunks]
        dst_hbm = refs[nchunks + 1]
        outs = refs[nchunks + 2 : 2 * nchunks + 2]
        src_v, dst_v, buf0, buf1, zbuf, acc, sem0, sem1 = refs[2 * nchunks + 2 :]

        c = lax.axis_index("c")
        s = lax.axis_index("s")
        u = c * NS + s
        _fill_zero(zbuf, DC)

        for ch in range(nchunks):
            def zcp(k, _):
                pltpu.sync_copy(zbuf, acc.at[pl.ds(s * RPT + k * ZROWS, ZROWS)])
                return 0

            lax.fori_loop(0, CPR, zcp, 0)
            plsc.subcore_barrier()

            table = tables[ch]

            def kblk(k, _):
                pltpu.sync_copy(src_hbm.at[u, pl.ds(k * KB, KB)], src_v)
                pltpu.sync_copy(dst_hbm.at[u, pl.ds(k * KB, KB)], dst_v)

                def estep(j2, _):
                    j0 = j2 * 2
                    cp0 = pltpu.make_async_copy(table.at[src_v.at[j0]], buf0, sem0)
                    cp0.start()
                    cp1 = pltpu.make_async_copy(
                        table.at[src_v.at[j0 + 1]], buf1, sem1)
                    cp1.start()
                    cp0.wait()
                    pltpu.sync_copy(buf0, acc.at[dst_v.at[j0]], add=True)
                    cp1.wait()
                    pltpu.sync_copy(buf1, acc.at[dst_v.at[j0 + 1]], add=True)
                    return 0

                lax.fori_loop(0, KB // 2, estep, 0)
                return 0

            lax.fori_loop(0, NB // KB, kblk, 0)
            plsc.subcore_barrier()
            pltpu.sync_copy(
                acc.at[pl.ds(s * CPO, CPO)],
                outs[ch].at[c, pl.ds(s * CPO, CPO)],
            )
            if ch + 1 < nchunks:
                plsc.subcore_barrier()

    return agg_kernel


_deg_kernel = _make_deg_kernel()
_agg2 = _make_agg_kernel(2)
_agg4 = _make_agg_kernel(4)


def _prep_body(dega_ref, degb_ref, x_ref, xs0_ref, xs1_ref, dinv_ref):
    deg = dega_ref[:, 0:1] + degb_ref[:, 0:1] + 1.0
    dinv = lax.rsqrt(deg)
    xs = x_ref[...] * dinv
    xs0_ref[...] = xs[:, :DC]
    xs1_ref[...] = xs[:, DC:]
    dinv_ref[...] = jnp.broadcast_to(dinv, (R, 8))


def _tc_prep(dega, degb, xpad):
    return pl.pallas_call(
        _prep_body,
        grid=(GRID,),
        in_specs=[
            pl.BlockSpec((R, 16), lambda i: (i, 0)),
            pl.BlockSpec((R, 16), lambda i: (i, 0)),
            pl.BlockSpec((R, 2 * DC), lambda i: (i, 0)),
        ],
        out_specs=[
            pl.BlockSpec((R, DC), lambda i: (i, 0)),
            pl.BlockSpec((R, DC), lambda i: (i, 0)),
            pl.BlockSpec((R, 8), lambda i: (i, 0)),
        ],
        out_shape=[
            jax.ShapeDtypeStruct((N, DC), jnp.float32),
            jax.ShapeDtypeStruct((N, DC), jnp.float32),
            jax.ShapeDtypeStruct((N, 8), jnp.float32),
        ],
    )(dega, degb, xpad)


def _mm1_body(e0_ref, e1_ref, xs0_ref, xs1_ref, dinv_ref, w_ref, b_ref,
              h0_ref, h1_ref, h2_ref, h3_ref):
    dinv = dinv_ref[:, 0:1]
    s0 = (e0_ref[0] + e0_ref[1] + xs0_ref[...])
    s1 = (e1_ref[0] + e1_ref[1] + xs1_ref[...])
    sfull = jnp.concatenate([s0, s1], axis=1) * dinv
    h = jnp.maximum(jnp.dot(sfull, w_ref[...],
                            preferred_element_type=jnp.float32)
                    + b_ref[...], 0.0)
    hs = h * dinv
    h0_ref[...] = hs[:, 0 * DC:1 * DC]
    h1_ref[...] = hs[:, 1 * DC:2 * DC]
    h2_ref[...] = hs[:, 2 * DC:3 * DC]
    h3_ref[...] = hs[:, 3 * DC:4 * DC]


def _tc_mm1(e0, e1, xs0, xs1, dinv, w1p, b1):
    es_spec = pl.BlockSpec((NC, R, DC), lambda i: (0, i, 0))
    row_spec = pl.BlockSpec((R, DC), lambda i: (i, 0))
    return pl.pallas_call(
        _mm1_body,
        grid=(GRID,),
        in_specs=[
            es_spec, es_spec, row_spec, row_spec,
            pl.BlockSpec((R, 8), lambda i: (i, 0)),
            pl.BlockSpec((2 * DC, 128), lambda i: (0, 0)),
            pl.BlockSpec((1, 128), lambda i: (0, 0)),
        ],
        out_specs=[row_spec, row_spec, row_spec, row_spec],
        out_shape=[jax.ShapeDtypeStruct((N, DC), jnp.float32) for _ in range(4)],
    )(e0, e1, xs0, xs1, dinv, w1p, b1)


def _mm2_body(e0_ref, e1_ref, e2_ref, e3_ref,
              h0_ref, h1_ref, h2_ref, h3_ref,
              dinv_ref, w2_ref, b2_ref, lo_ref, hi_ref,
              wl_ref, bl_ref, out_ref, pooled):
    i = pl.program_id(0)
    dinv = dinv_ref[:, 0:1]
    parts = []
    for e_ref, h_ref in ((e0_ref, h0_ref), (e1_ref, h1_ref),
                         (e2_ref, h2_ref), (e3_ref, h3_ref)):
        parts.append(e_ref[0] + e_ref[1] + h_ref[...])
    sfull = jnp.concatenate(parts, axis=1) * dinv
    h = jnp.maximum(jnp.dot(sfull, w2_ref[...],
                            preferred_element_type=jnp.float32)
                    + b2_ref[...], 0.0)
    rows = i * R + lax.broadcasted_iota(jnp.int32, (R, 1), 0)
    member = ((rows >= lo_ref[...]) & (rows < hi_ref[...])).astype(jnp.float32)
    contrib = lax.dot_general(member, h, (((0,), (0,)), ((), ())),
                              preferred_element_type=jnp.float32)

    @pl.when(i == 0)
    def _():
        pooled[...] = contrib

    @pl.when(i > 0)
    def _():
        pooled[...] = pooled[...] + contrib

    @pl.when(i == GRID - 1)
    def _():
        logits = jnp.dot(pooled[...], wl_ref[...],
                         preferred_element_type=jnp.float32) + bl_ref[...]
        mask = lax.broadcasted_iota(jnp.int32, (1, 128), 1) < 10
        neg = jnp.float32(-1e30)
        lm = jnp.where(mask, logits, neg)
        m = jnp.max(lm, axis=1, keepdims=True)
        ex = jnp.where(mask, jnp.exp(lm - m), 0.0)
        lse = jnp.log(jnp.sum(ex, axis=1, keepdims=True))
        out_ref[...] = lm - m - lse


def _tc_mm2(es2, h1s, dinv, w2, b2, lo, hi, wlp, blp):
    es_spec = pl.BlockSpec((NC, R, DC), lambda i: (0, i, 0))
    row_spec = pl.BlockSpec((R, DC), lambda i: (i, 0))
    full = lambda a, b: pl.BlockSpec((a, b), lambda i: (0, 0))
    return pl.pallas_call(
        _mm2_body,
        grid=(GRID,),
        in_specs=[
            es_spec, es_spec, es_spec, es_spec,
            row_spec, row_spec, row_spec, row_spec,
            pl.BlockSpec((R, 8), lambda i: (i, 0)),
            full(128, 256), full(1, 256),
            full(1, NGRAPH), full(1, NGRAPH),
            full(256, 128), full(1, 128),
        ],
        out_specs=pl.BlockSpec((NGRAPH, 128), lambda i: (0, 0)),
        out_shape=jax.ShapeDtypeStruct((NGRAPH, 128), jnp.float32),
        scratch_shapes=[pltpu.VMEM((NGRAPH, 256), jnp.float32)],
    )(*es2, *h1s, dinv, w2, b2, lo, hi, wlp, blp)


def kernel(x, edge_index, ptr, W1, b1, W2, b2, Wl, bl):
    src = edge_index[0]
    dst = edge_index[1]
    srcp = jnp.concatenate(
        [src, jnp.zeros((PADE,), jnp.int32)]).reshape(NW, NB, BATCH)
    dstp = jnp.concatenate(
        [dst, jnp.full((PADE,), N, jnp.int32)]).reshape(NW, NB, BATCH)

    deg = _deg_kernel(dstp)

    xpad = jnp.pad(x, ((0, 0), (0, 2 * DC - x.shape[1])))
    xs0, xs1, dinv = _tc_prep(deg[0], deg[1], xpad)

    es1 = _agg2(xs0, xs1, srcp, dstp)

    w1p = jnp.pad(W1, ((0, 2 * DC - W1.shape[0]), (0, 0)))
    h1s = _tc_mm1(es1[0], es1[1], xs0, xs1, dinv, w1p, b1.reshape(1, 128))

    es2 = _agg4(*h1s, srcp, dstp)

    lo = ptr[:NGRAPH].reshape(1, NGRAPH)
    hi = ptr[1:NGRAPH + 1].reshape(1, NGRAPH)
    wlp = jnp.pad(Wl, ((0, 0), (0, 128 - Wl.shape[1])))
    blp = jnp.pad(bl, (0, 128 - bl.shape[0])).reshape(1, 128)
    out = _tc_mm2(es2, h1s, dinv, W2, b2.reshape(1, 256), lo, hi, wlp, blp)
    return out[:, :10]
